# baseline (speedup 1.0000x reference)
import functools

import jax
import jax.numpy as jnp
from jax import lax
from jax.experimental import pallas as pl
from jax.experimental.pallas import tpu as pltpu
from jax.experimental.pallas import tpu_sc as plsc

NC, NS, LANES = 2, 16, 16
NW = NC * NS
B, L, T, EMB = 1024, 50, 20, 64
TOKENS = B * L
TPW = TOKENS // NW
CT = 64
CHUNKS = TPW // CT
CIDX = CT * T
IPW = TPW * T
NJ = EMB // LANES
NH = CT // LANES

_GATHER_DNUMS = lax.GatherDimensionNumbers(
    offset_dims=(), collapsed_slice_dims=(0,), start_index_map=(0,))


@functools.partial(
    pl.kernel,
    out_type=jax.ShapeDtypeStruct((TOKENS * EMB,), jnp.float32),
    mesh=plsc.VectorSubcoreMesh(
        core_axis_name="c", subcore_axis_name="s",
        num_cores=NC, num_subcores=NS),
    scratch_types=[
        pltpu.VMEM((IPW + LANES,), jnp.int32),
        pltpu.VMEM((2, T, CT), jnp.int32),
        pltpu.VMEM((2, CT, EMB), jnp.bfloat16),
        pltpu.VMEM((2, CT * EMB), jnp.float32),
        pltpu.VMEM((1, EMB), jnp.bfloat16),
        pltpu.SemaphoreType.DMA,
        pltpu.SemaphoreType.DMA,
        pltpu.SemaphoreType.DMA,
        pltpu.SemaphoreType.DMA,
    ],
    compiler_params=pltpu.CompilerParams(
        use_tc_tiling_on_sc=False, needs_layout_passes=False),
)
def _sc_kernel(seq_hbm, w_hbm, out_hbm, idx_all, idx_m, acc_v, out_v,
               w0_v, gsem0, gsem1, osem0, osem1):
    wid = lax.axis_index("s") * NC + lax.axis_index("c")
    last_lane = jnp.full((LANES,), LANES - 1, jnp.int32)
    stride_v = lax.iota(jnp.int32, LANES) * T
    iota2 = lax.iota(jnp.int32, LANES) * 2
    zero_b = jnp.zeros((2 * LANES,), jnp.bfloat16)
    hi_mask = jnp.full((LANES,), -65536, jnp.int32)
    gsems = (gsem0, gsem1)
    osems = (osem0, osem1)

    pltpu.sync_copy(seq_hbm.at[pl.ds(wid * IPW, IPW)],
                    idx_all.at[pl.ds(0, IPW)])
    pltpu.sync_copy(w_hbm.at[pl.ds(0, 1)], w0_v)
    w0e, w0o = [], []
    for g in range(NJ // 2):
        xi = plsc.bitcast(w0_v[0, pl.ds(g * 2 * LANES, 2 * LANES)], jnp.int32)
        w0e.append(plsc.bitcast(xi << 16, jnp.float32))
        w0o.append(plsc.bitcast(xi & hi_mask, jnp.float32))

    def remap_and_fire(cn, par):
        @pl.loop(0, CT)
        def _z(k):
            for g in range(NJ // 2):
                acc_v[par, k, pl.ds(g * 2 * LANES, 2 * LANES)] = zero_b
        for t in range(T):
            for h in range(NH):
                pos = stride_v + (cn * CIDX + h * LANES * T + t)
                v = plsc.load_gather(idx_all, [pos])
                idx_m[par, t, pl.ds(h * LANES, LANES)] = jnp.maximum(v - 1, 0)
        for t in range(T):
            pltpu.async_copy(w_hbm.at[idx_m.at[par, t]],
                             acc_v.at[par], gsems[par], add=True)

    def process(cur, par):
        for _ in range(T):
            pltpu.make_async_copy(w_hbm.at[pl.ds(0, CT)],
                                  acc_v.at[par], gsems[par]).wait()
        @pl.when(cur >= 2)
        def _():
            pltpu.make_async_copy(out_hbm.at[pl.ds(0, CT * EMB)],
                                  out_v.at[par], osems[par]).wait()

        @pl.loop(0, CT, unroll=2)
        def _tok(k):
            base_r = k * T
            flat = cur * CIDX + base_r
            v1 = idx_all[pl.ds(flat, LANES)]
            v2 = idx_all[pl.ds(flat + LANES, LANES)]
            one_i = jnp.ones((LANES,), jnp.int32)
            zero_i = jnp.zeros((LANES,), jnp.int32)
            lanes_k = lax.iota(jnp.int32, LANES)
            ind = (jnp.where(v1 == 0, one_i, zero_i)
                   + jnp.where((v2 == 0) & (lanes_k < (T - LANES)),
                               one_i, zero_i))
            cs = jnp.cumsum(ind)
            nz = lax.gather(
                cs, last_lane[:, None], _GATHER_DNUMS, slice_sizes=(1,),
                mode=lax.GatherScatterMode.PROMISE_IN_BOUNDS,
            ).astype(jnp.float32)
            cnt = jnp.float32(T) - nz
            pos = cnt > 0.0
            scale = jnp.where(pos, 1.0 / jnp.where(pos, cnt, 1.0), 0.0)
            for g in range(NJ // 2):
                ai = plsc.bitcast(
                    acc_v[par, k, pl.ds(g * 2 * LANES, 2 * LANES)], jnp.int32)
                alo = plsc.bitcast(ai << 16, jnp.float32)
                ahi = plsc.bitcast(ai & hi_mask, jnp.float32)
                base_o = k * EMB + g * 2 * LANES
                plsc.store_scatter(out_v.at[par], [iota2 + base_o],
                                   (alo - nz * w0e[g]) * scale)
                plsc.store_scatter(out_v.at[par], [iota2 + (base_o + 1)],
                                   (ahi - nz * w0o[g]) * scale)

        pltpu.async_copy(out_v.at[par],
                         out_hbm.at[pl.ds((wid * TPW + cur * CT) * EMB,
                                          CT * EMB)],
                         osems[par])

    remap_and_fire(0, 0)

    @pl.loop(0, CHUNKS, step=2)
    def _body(c):
        for b in range(2):
            cur = c + b
            nxt_par = 1 - b

            @pl.when(cur < CHUNKS)
            def _():
                @pl.when(cur + 1 < CHUNKS)
                def _():
                    remap_and_fire(cur + 1, nxt_par)

                process(cur, b)

    for par in range(2):
        pltpu.make_async_copy(out_hbm.at[pl.ds(0, CT * EMB)],
                              out_v.at[par], osems[par]).wait()


def kernel(seq, W):
    out = _sc_kernel(seq.reshape(-1), W.astype(jnp.bfloat16))
    return out.reshape(B, L, EMB)


# speedup vs baseline: 1.1181x; 1.1181x over previous
import functools

import jax
import jax.numpy as jnp
from jax import lax
from jax.experimental import pallas as pl
from jax.experimental.pallas import tpu as pltpu
from jax.experimental.pallas import tpu_sc as plsc

NC, NS, LANES = 2, 16, 16
NW = NC * NS
B, L, T, EMB = 1024, 50, 20, 64
TOKENS = B * L
TPW = TOKENS // NW
CT = 64
CHUNKS = TPW // CT
CIDX = CT * T
IPW = TPW * T
NJ = EMB // LANES
NH = CT // LANES

_GATHER_DNUMS = lax.GatherDimensionNumbers(
    offset_dims=(), collapsed_slice_dims=(0,), start_index_map=(0,))


@functools.partial(
    pl.kernel,
    out_type=jax.ShapeDtypeStruct((TOKENS * EMB,), jnp.float32),
    mesh=plsc.VectorSubcoreMesh(
        core_axis_name="c", subcore_axis_name="s",
        num_cores=NC, num_subcores=NS),
    scratch_types=[
        pltpu.VMEM((IPW + LANES,), jnp.int32),
        pltpu.VMEM((2, T, CT), jnp.int32),
        pltpu.VMEM((2, CT, EMB), jnp.float32),
        pltpu.VMEM((2, CT * EMB), jnp.float32),
        pltpu.VMEM((1, EMB), jnp.float32),
        pltpu.SemaphoreType.DMA,
        pltpu.SemaphoreType.DMA,
        pltpu.SemaphoreType.DMA,
        pltpu.SemaphoreType.DMA,
    ],
    compiler_params=pltpu.CompilerParams(
        use_tc_tiling_on_sc=False, needs_layout_passes=False),
)
def _sc_kernel(seq_hbm, w_hbm, out_hbm, idx_all, idx_m, acc_v, out_v,
               w0_v, gsem0, gsem1, osem0, osem1):
    wid = lax.axis_index("s") * NC + lax.axis_index("c")
    last_lane = jnp.full((LANES,), LANES - 1, jnp.int32)
    stride_v = lax.iota(jnp.int32, LANES) * T
    zero_f = jnp.zeros((LANES,), jnp.float32)
    gsems = (gsem0, gsem1)
    osems = (osem0, osem1)

    pltpu.sync_copy(seq_hbm.at[pl.ds(wid * IPW, IPW)],
                    idx_all.at[pl.ds(0, IPW)])
    pltpu.sync_copy(w_hbm.at[pl.ds(0, 1)], w0_v)
    w0 = [w0_v[0, pl.ds(j * LANES, LANES)] for j in range(NJ)]

    def remap_and_fire(cn, par):
        @pl.loop(0, CT)
        def _z(k):
            for j in range(NJ):
                acc_v[par, k, pl.ds(j * LANES, LANES)] = zero_f
        for t in range(T):
            for h in range(NH):
                pos = stride_v + (cn * CIDX + h * LANES * T + t)
                v = plsc.load_gather(idx_all, [pos])
                idx_m[par, t, pl.ds(h * LANES, LANES)] = jnp.maximum(v - 1, 0)
        for t in range(T):
            pltpu.async_copy(w_hbm.at[idx_m.at[par, t]],
                             acc_v.at[par], gsems[par], add=True)

    def process(cur, par):
        for _ in range(T):
            pltpu.make_async_copy(w_hbm.at[pl.ds(0, CT)],
                                  acc_v.at[par], gsems[par]).wait()
        @pl.when(cur >= 2)
        def _():
            pltpu.make_async_copy(out_hbm.at[pl.ds(0, CT * EMB)],
                                  out_v.at[par], osems[par]).wait()

        @pl.loop(0, CT, unroll=2)
        def _tok(k):
            base_r = k * T
            flat = cur * CIDX + base_r
            v1 = idx_all[pl.ds(flat, LANES)]
            v2 = idx_all[pl.ds(flat + LANES, LANES)]
            one_i = jnp.ones((LANES,), jnp.int32)
            zero_i = jnp.zeros((LANES,), jnp.int32)
            lanes_k = lax.iota(jnp.int32, LANES)
            ind = (jnp.where(v1 == 0, one_i, zero_i)
                   + jnp.where((v2 == 0) & (lanes_k < (T - LANES)),
                               one_i, zero_i))
            cs = jnp.cumsum(ind)
            nz = lax.gather(
                cs, last_lane[:, None], _GATHER_DNUMS, slice_sizes=(1,),
                mode=lax.GatherScatterMode.PROMISE_IN_BOUNDS,
            ).astype(jnp.float32)
            cnt = jnp.float32(T) - nz
            pos = cnt > 0.0
            scale = jnp.where(pos, 1.0 / jnp.where(pos, cnt, 1.0), 0.0)
            for j in range(NJ):
                a = acc_v[par, k, pl.ds(j * LANES, LANES)]
                out_v[par, pl.ds(k * EMB + j * LANES, LANES)] = \
                    (a - nz * w0[j]) * scale

        pltpu.async_copy(out_v.at[par],
                         out_hbm.at[pl.ds((wid * TPW + cur * CT) * EMB,
                                          CT * EMB)],
                         osems[par])

    remap_and_fire(0, 0)

    @pl.loop(0, CHUNKS, step=2)
    def _body(c):
        for b in range(2):
            cur = c + b
            nxt_par = 1 - b

            @pl.when(cur < CHUNKS)
            def _():
                @pl.when(cur + 1 < CHUNKS)
                def _():
                    remap_and_fire(cur + 1, nxt_par)

                process(cur, b)

    for par in range(2):
        pltpu.make_async_copy(out_hbm.at[pl.ds(0, CT * EMB)],
                              out_v.at[par], osems[par]).wait()


def kernel(seq, W):
    out = _sc_kernel(seq.reshape(-1), W)
    return out.reshape(B, L, EMB)
